# Initial kernel scaffold; baseline (speedup 1.0000x reference)
import jax
import jax.numpy as jnp
from jax.experimental import pallas as pl


def kernel(outputs, group_indices):
    raise NotImplementedError("write your pallas kernel here")


# speedup vs baseline: 1.2332x; 1.2332x over previous
import functools

import jax
import jax.numpy as jnp
from jax import lax
from jax.experimental import pallas as pl
from jax.experimental.pallas import tpu as pltpu
from jax.experimental.pallas import tpu_sc as plsc

N_FIELDS = 26
BATCH = 16384
NUM_CORES = 2
NUM_SUBCORES = 16
NW = NUM_CORES * NUM_SUBCORES
BPW = BATCH // NW
LANES = 16


@functools.partial(
    pl.kernel,
    mesh=plsc.VectorSubcoreMesh(core_axis_name="c", subcore_axis_name="s"),
    out_type=jax.ShapeDtypeStruct((BATCH,), jnp.float32),
    scratch_types=[
        pltpu.VMEM((BPW,), jnp.int32),
        pltpu.VMEM((BPW,), jnp.int32),
        pltpu.VMEM((BPW,), jnp.float32),
        pltpu.SemaphoreType.DMA,
    ],
)
def _sc_gather(flat_hbm, idx_hbm, out_hbm, idx_v, lin_v, rows_v, sem):
    wid = lax.axis_index("s") * NUM_CORES + lax.axis_index("c")
    base = wid * BPW
    pltpu.sync_copy(idx_hbm.at[pl.ds(base, BPW)], idx_v)
    for i in range(BPW // LANES):
        fld = idx_v[pl.ds(i * LANES, LANES)]
        pos = base + i * LANES + lax.iota(jnp.int32, 16)
        lin_v[pl.ds(i * LANES, LANES)] = fld * BATCH + pos
    pltpu.async_copy(flat_hbm.at[lin_v], rows_v, sem).wait()
    pltpu.sync_copy(rows_v, out_hbm.at[pl.ds(base, BPW)])


def kernel(outputs, group_indices):
    flat = outputs.reshape(N_FIELDS * BATCH)
    idx = group_indices.astype(jnp.int32).reshape(BATCH)
    out = _sc_gather(flat, idx)
    return out.reshape(BATCH, 1)


# speedup vs baseline: 1.2535x; 1.0164x over previous
import functools

import jax
import jax.numpy as jnp
from jax import lax
from jax.experimental import pallas as pl
from jax.experimental.pallas import tpu as pltpu
from jax.experimental.pallas import tpu_sc as plsc

N_FIELDS = 26
BATCH = 16384
NUM_CORES = 2
NUM_SUBCORES = 16
NW = NUM_CORES * NUM_SUBCORES
BPW = BATCH // NW
LANES = 16


@functools.partial(
    pl.kernel,
    mesh=plsc.VectorSubcoreMesh(core_axis_name="c", subcore_axis_name="s"),
    out_type=jax.ShapeDtypeStruct((BATCH,), jnp.float32),
    scratch_types=[
        pltpu.VMEM((BPW,), jnp.int32),
        pltpu.VMEM((BPW,), jnp.int32),
        pltpu.VMEM((BPW,), jnp.float32),
        pltpu.SemaphoreType.DMA,
    ],
)
def _sc_gather(flat_hbm, idx_hbm, out_hbm, idx_v, lin_v, rows_v, sem):
    wid = lax.axis_index("s") * NUM_CORES + lax.axis_index("c")
    base = wid * BPW
    pltpu.sync_copy(idx_hbm.at[pl.ds(base, BPW)], idx_v)

    def body(i, carry):
        off = i * LANES
        fld = idx_v[pl.ds(off, LANES)]
        pos = base + off + lax.iota(jnp.int32, 16)
        lin_v[pl.ds(off, LANES)] = fld * BATCH + pos
        return carry

    lax.fori_loop(0, BPW // LANES, body, 0)
    pltpu.async_copy(flat_hbm.at[lin_v], rows_v, sem).wait()
    pltpu.sync_copy(rows_v, out_hbm.at[pl.ds(base, BPW)])


def kernel(outputs, group_indices):
    flat = outputs.reshape(N_FIELDS * BATCH)
    idx = group_indices.astype(jnp.int32).reshape(BATCH)
    out = _sc_gather(flat, idx)
    return out.reshape(BATCH, 1)
